# Initial kernel scaffold; baseline (speedup 1.0000x reference)
import jax
import jax.numpy as jnp
from jax.experimental import pallas as pl


def kernel(relative_position_bias_table, relative_position_index):
    raise NotImplementedError("write your pallas kernel here")


# speedup vs baseline: 5.3695x; 5.3695x over previous
import functools

import jax
import jax.numpy as jnp
from jax import lax
from jax.experimental import pallas as pl
from jax.experimental.pallas import tpu as pltpu
from jax.experimental.pallas import tpu_sc as plsc

HEADS = 16
SEQ = 1025
NREL = 3972
Q = SEQ * SEQ
QM = Q - 1
PART = QM // 4
CH = 8208
CHUNKS = PART // CH
GPC = CH // 16
TCOL = 3976
IDX_LEN = 8 + Q + 23


@functools.partial(
    pl.kernel,
    out_type=jax.ShapeDtypeStruct((HEADS * Q,), jnp.float32),
    mesh=plsc.VectorSubcoreMesh(core_axis_name="c", subcore_axis_name="s"),
    compiler_params=pltpu.CompilerParams(needs_layout_passes=False),
    scratch_types=[
        pltpu.VMEM((2 * TCOL,), jnp.float32),
        pltpu.VMEM((2 * TCOL,), jnp.float32),
        pltpu.VMEM((CH + 24,), jnp.int32),
        pltpu.VMEM((CH,), jnp.float32),
        pltpu.VMEM((CH,), jnp.float32),
        pltpu.VMEM((16,), jnp.int32),
        pltpu.VMEM((24,), jnp.int32),
    ],
)
def _rpb_sc(table_hbm, idx_hbm, out_hbm, tw1, tw2, idx_v, oa, ob, tail_v, t8_v):
    cid = lax.axis_index("c")
    sid = lax.axis_index("s")
    wid = sid * 2 + cid
    r = wid % 8
    e = wid // 8
    hA = r
    hB = r + 8
    s = r
    off = 8 - s

    pltpu.sync_copy(table_hbm.at[pl.ds(r * TCOL, 2 * TCOL)], tw1)
    pltpu.sync_copy(table_hbm.at[pl.ds((r + 8) * TCOL, 2 * TCOL)], tw2)
    pltpu.sync_copy(idx_hbm.at[pl.ds(Q - 1, 16)], tail_v)

    def chunk_body(k, carry):
        base = e * PART + k * CH
        pltpu.sync_copy(idx_hbm.at[pl.ds(base, CH + 24)], idx_v)

        def g_body(g, c2):
            iv = idx_v[pl.ds(off + g * 16, 16)]
            oa[pl.ds(g * 16, 16)] = plsc.load_gather(tw1, [iv + TCOL])
            ob[pl.ds(g * 16, 16)] = plsc.load_gather(tw2, [iv + TCOL])
            return c2

        lax.fori_loop(0, GPC, g_body, 0, unroll=False)

        @pl.when((e == 0) & (k == 0) & (s > 0))
        def _():
            lane = lax.iota(jnp.int32, 16)
            iv0 = idx_v[pl.ds(off, 16)]
            tpos = jnp.minimum(9 - s + lane, 15)
            tidx = plsc.load_gather(tail_v, [tpos])
            m = lane < s
            nmA = plsc.load_gather(tw1, [iv0 + TCOL])
            nmB = plsc.load_gather(tw2, [iv0 + TCOL])
            spA = plsc.load_gather(tw1, [tidx])
            spB = plsc.load_gather(tw2, [tidx])
            oa[pl.ds(0, 16)] = jnp.where(m, spA, nmA)
            ob[pl.ds(0, 16)] = jnp.where(m, spB, nmB)

        pltpu.sync_copy(
            oa, out_hbm.at[pl.ds(pl.multiple_of(hA * Q + base - s, 8), CH)]
        )
        pltpu.sync_copy(
            ob, out_hbm.at[pl.ds(pl.multiple_of(hB * Q + base - s, 8), CH)]
        )
        return carry

    lax.fori_loop(0, CHUNKS, chunk_body, 0, unroll=False)

    @pl.when((e == 3) & (s == 7))
    def _():
        pltpu.sync_copy(idx_hbm.at[pl.ds(QM, 24)], t8_v)
        iv = t8_v[pl.ds(1, 16)]
        oa[pl.ds(0, 16)] = plsc.load_gather(tw1, [iv + TCOL])
        ob[pl.ds(0, 16)] = plsc.load_gather(tw2, [iv + TCOL])
        pltpu.sync_copy(
            oa.at[pl.ds(0, 8)],
            out_hbm.at[pl.ds(pl.multiple_of(hA * Q + QM - 7, 8), 8)],
        )
        pltpu.sync_copy(
            ob.at[pl.ds(0, 8)],
            out_hbm.at[pl.ds(pl.multiple_of(hB * Q + QM - 7, 8), 8)],
        )


def kernel(relative_position_bias_table, relative_position_index):
    tpad = (
        jnp.zeros((HEADS + 1, TCOL), jnp.float32)
        .at[1:, :NREL]
        .set(relative_position_bias_table.T)
        .reshape(-1)
    )
    idx2 = (
        jnp.zeros((IDX_LEN,), jnp.int32)
        .at[8 : 8 + Q]
        .set(relative_position_index.reshape(-1))
    )
    out = _rpb_sc(tpad, idx2)
    return out.reshape(HEADS, SEQ, SEQ)


# speedup vs baseline: 5.6064x; 1.0441x over previous
import functools

import jax
import jax.numpy as jnp
from jax import lax
from jax.experimental import pallas as pl
from jax.experimental.pallas import tpu as pltpu
from jax.experimental.pallas import tpu_sc as plsc

HEADS = 16
SEQ = 1025
NREL = 3972
Q = SEQ * SEQ
QM = Q - 1
PART = QM // 4
CH = 8208
CHUNKS = PART // CH
GPC = CH // 16
TCOL = 3976
IDX_LEN = 8 + Q + 23


@functools.partial(
    pl.kernel,
    out_type=jax.ShapeDtypeStruct((HEADS * Q,), jnp.float32),
    mesh=plsc.VectorSubcoreMesh(core_axis_name="c", subcore_axis_name="s"),
    compiler_params=pltpu.CompilerParams(needs_layout_passes=False),
    scratch_types=[
        pltpu.VMEM((2 * TCOL,), jnp.float32),
        pltpu.VMEM((2 * TCOL,), jnp.float32),
        pltpu.VMEM((CH + 24,), jnp.int32),
        pltpu.VMEM((CH,), jnp.float32),
        pltpu.VMEM((CH,), jnp.float32),
        pltpu.VMEM((16,), jnp.int32),
        pltpu.VMEM((24,), jnp.int32),
    ],
)
def _rpb_sc(table_hbm, idx_hbm, out_hbm, tw1, tw2, idx_v, oa, ob, tail_v, t8_v):
    cid = lax.axis_index("c")
    sid = lax.axis_index("s")
    wid = sid * 2 + cid
    r = wid % 8
    e = wid // 8
    hA = r
    hB = r + 8
    s = r
    off = 8 - s

    pltpu.sync_copy(table_hbm.at[pl.ds(r * TCOL, 2 * TCOL)], tw1)
    pltpu.sync_copy(table_hbm.at[pl.ds((r + 8) * TCOL, 2 * TCOL)], tw2)
    pltpu.sync_copy(idx_hbm.at[pl.ds(Q - 1, 16)], tail_v)

    def chunk_body(k, carry):
        base = e * PART + k * CH
        pltpu.sync_copy(idx_hbm.at[pl.ds(base, CH + 24)], idx_v)

        def g_body(g, c2):
            iv = idx_v[pl.ds(off + g * 16, 16)]
            oa[pl.ds(g * 16, 16)] = plsc.load_gather(tw1, [iv + TCOL])
            ob[pl.ds(g * 16, 16)] = plsc.load_gather(tw2, [iv + TCOL])
            return c2

        lax.fori_loop(0, GPC, g_body, 0, unroll=9)

        @pl.when((e == 0) & (k == 0) & (s > 0))
        def _():
            lane = lax.iota(jnp.int32, 16)
            iv0 = idx_v[pl.ds(off, 16)]
            tpos = jnp.minimum(9 - s + lane, 15)
            tidx = plsc.load_gather(tail_v, [tpos])
            m = lane < s
            nmA = plsc.load_gather(tw1, [iv0 + TCOL])
            nmB = plsc.load_gather(tw2, [iv0 + TCOL])
            spA = plsc.load_gather(tw1, [tidx])
            spB = plsc.load_gather(tw2, [tidx])
            oa[pl.ds(0, 16)] = jnp.where(m, spA, nmA)
            ob[pl.ds(0, 16)] = jnp.where(m, spB, nmB)

        pltpu.sync_copy(
            oa, out_hbm.at[pl.ds(pl.multiple_of(hA * Q + base - s, 8), CH)]
        )
        pltpu.sync_copy(
            ob, out_hbm.at[pl.ds(pl.multiple_of(hB * Q + base - s, 8), CH)]
        )
        return carry

    lax.fori_loop(0, CHUNKS, chunk_body, 0, unroll=False)

    @pl.when((e == 3) & (s == 7))
    def _():
        pltpu.sync_copy(idx_hbm.at[pl.ds(QM, 24)], t8_v)
        iv = t8_v[pl.ds(1, 16)]
        oa[pl.ds(0, 16)] = plsc.load_gather(tw1, [iv + TCOL])
        ob[pl.ds(0, 16)] = plsc.load_gather(tw2, [iv + TCOL])
        pltpu.sync_copy(
            oa.at[pl.ds(0, 8)],
            out_hbm.at[pl.ds(pl.multiple_of(hA * Q + QM - 7, 8), 8)],
        )
        pltpu.sync_copy(
            ob.at[pl.ds(0, 8)],
            out_hbm.at[pl.ds(pl.multiple_of(hB * Q + QM - 7, 8), 8)],
        )


def kernel(relative_position_bias_table, relative_position_index):
    tpad = (
        jnp.zeros((HEADS + 1, TCOL), jnp.float32)
        .at[1:, :NREL]
        .set(relative_position_bias_table.T)
        .reshape(-1)
    )
    idx2 = (
        jnp.zeros((IDX_LEN,), jnp.int32)
        .at[8 : 8 + Q]
        .set(relative_position_index.reshape(-1))
    )
    out = _rpb_sc(tpad, idx2)
    return out.reshape(HEADS, SEQ, SEQ)


# speedup vs baseline: 7.4624x; 1.3310x over previous
import functools

import jax
import jax.numpy as jnp
from jax import lax
from jax.experimental import pallas as pl
from jax.experimental.pallas import tpu as pltpu
from jax.experimental.pallas import tpu_sc as plsc

HEADS = 16
SEQ = 1025
NREL = 3972
Q = SEQ * SEQ
QM = Q - 1
PART = QM // 4
CH = 8208
CHUNKS = PART // CH
GPC = CH // 16
TCOL = 3976
IDX_LEN = 8 + Q + 23


@functools.partial(
    pl.kernel,
    out_type=jax.ShapeDtypeStruct((HEADS * Q,), jnp.float32),
    mesh=plsc.VectorSubcoreMesh(core_axis_name="c", subcore_axis_name="s"),
    compiler_params=pltpu.CompilerParams(needs_layout_passes=False),
    scratch_types=[
        pltpu.VMEM((2 * TCOL,), jnp.float32),
        pltpu.VMEM((2 * TCOL,), jnp.float32),
        pltpu.VMEM((CH + 24,), jnp.int32),
        pltpu.VMEM((CH,), jnp.float32),
        pltpu.VMEM((CH,), jnp.float32),
        pltpu.VMEM((16,), jnp.int32),
        pltpu.VMEM((24,), jnp.int32),
    ],
)
def _rpb_sc(table_hbm, idx_hbm, out_hbm, tw1, tw2, idx_v, oa, ob, tail_v, t8_v):
    cid = lax.axis_index("c")
    sid = lax.axis_index("s")
    wid = sid * 2 + cid
    r = wid % 8
    e = wid // 8
    hA = r
    hB = r + 8
    s = r
    off = 8 - s

    pltpu.sync_copy(table_hbm.at[pl.ds(r * TCOL, 2 * TCOL)], tw1)
    pltpu.sync_copy(table_hbm.at[pl.ds((r + 8) * TCOL, 2 * TCOL)], tw2)
    pltpu.sync_copy(idx_hbm.at[pl.ds(Q - 1, 16)], tail_v)

    def chunk_body(k, carry):
        base = e * PART + k * CH
        pltpu.sync_copy(idx_hbm.at[pl.ds(base, CH + 24)], idx_v)

        @plsc.parallel_loop(0, GPC, unroll=8)
        def g_body(g):
            iv = idx_v[pl.ds(off + g * 16, 16)]
            oa[pl.ds(g * 16, 16)] = plsc.load_gather(tw1, [iv + TCOL])
            ob[pl.ds(g * 16, 16)] = plsc.load_gather(tw2, [iv + TCOL])

        @pl.when((e == 0) & (k == 0) & (s > 0))
        def _():
            lane = lax.iota(jnp.int32, 16)
            iv0 = idx_v[pl.ds(off, 16)]
            tpos = jnp.minimum(9 - s + lane, 15)
            tidx = plsc.load_gather(tail_v, [tpos])
            m = lane < s
            nmA = plsc.load_gather(tw1, [iv0 + TCOL])
            nmB = plsc.load_gather(tw2, [iv0 + TCOL])
            spA = plsc.load_gather(tw1, [tidx])
            spB = plsc.load_gather(tw2, [tidx])
            oa[pl.ds(0, 16)] = jnp.where(m, spA, nmA)
            ob[pl.ds(0, 16)] = jnp.where(m, spB, nmB)

        pltpu.sync_copy(
            oa, out_hbm.at[pl.ds(pl.multiple_of(hA * Q + base - s, 8), CH)]
        )
        pltpu.sync_copy(
            ob, out_hbm.at[pl.ds(pl.multiple_of(hB * Q + base - s, 8), CH)]
        )
        return carry

    lax.fori_loop(0, CHUNKS, chunk_body, 0, unroll=False)

    @pl.when((e == 3) & (s == 7))
    def _():
        pltpu.sync_copy(idx_hbm.at[pl.ds(QM, 24)], t8_v)
        iv = t8_v[pl.ds(1, 16)]
        oa[pl.ds(0, 16)] = plsc.load_gather(tw1, [iv + TCOL])
        ob[pl.ds(0, 16)] = plsc.load_gather(tw2, [iv + TCOL])
        pltpu.sync_copy(
            oa.at[pl.ds(0, 8)],
            out_hbm.at[pl.ds(pl.multiple_of(hA * Q + QM - 7, 8), 8)],
        )
        pltpu.sync_copy(
            ob.at[pl.ds(0, 8)],
            out_hbm.at[pl.ds(pl.multiple_of(hB * Q + QM - 7, 8), 8)],
        )


def kernel(relative_position_bias_table, relative_position_index):
    tpad = (
        jnp.zeros((HEADS + 1, TCOL), jnp.float32)
        .at[1:, :NREL]
        .set(relative_position_bias_table.T)
        .reshape(-1)
    )
    idx2 = (
        jnp.zeros((IDX_LEN,), jnp.int32)
        .at[8 : 8 + Q]
        .set(relative_position_index.reshape(-1))
    )
    out = _rpb_sc(tpad, idx2)
    return out.reshape(HEADS, SEQ, SEQ)


# speedup vs baseline: 17.2476x; 2.3113x over previous
import functools

import jax
import jax.numpy as jnp
from jax import lax
from jax.experimental import pallas as pl
from jax.experimental.pallas import tpu as pltpu
from jax.experimental.pallas import tpu_sc as plsc

HEADS = 16
SEQ = 1025
NREL = 3972
ROWS_PAD = 1032
ROW_LEN = 1040
RG = ROWS_PAD // 8
NPAIR = HEADS // 2
UNITS = NPAIR * RG
NW = 32
UNITS_PER_W = -(-UNITS // NW)
GROUPS = ROW_LEN // 16


@functools.partial(
    pl.kernel,
    out_type=jax.ShapeDtypeStruct((HEADS, ROWS_PAD, ROW_LEN), jnp.float32),
    mesh=plsc.VectorSubcoreMesh(core_axis_name="c", subcore_axis_name="s"),
    compiler_params=pltpu.CompilerParams(needs_layout_passes=False),
    scratch_types=[
        pltpu.VMEM((HEADS * NREL,), jnp.float32),
        pltpu.VMEM((8 * ROW_LEN,), jnp.int32),
        pltpu.VMEM((8, ROW_LEN), jnp.float32),
        pltpu.VMEM((8, ROW_LEN), jnp.float32),
    ],
)
def _rpb_sc(table_hbm, idx_hbm, out_hbm, table_v, idx_v, oa, ob):
    cid = lax.axis_index("c")
    sid = lax.axis_index("s")
    wid = sid * 2 + cid

    pltpu.sync_copy(table_hbm, table_v)

    def unit_body(t, carry):
        u = wid + NW * t

        @pl.when(u < UNITS)
        def _():
            hp = u % NPAIR
            rg = u // NPAIR
            bA = hp * NREL
            bB = (hp + 8) * NREL
            pltpu.sync_copy(
                idx_hbm.at[pl.ds(rg * (8 * ROW_LEN), 8 * ROW_LEN)], idx_v
            )
            for row in range(8):

                @plsc.parallel_loop(0, GROUPS, unroll=8)
                def _g(c):
                    iv = idx_v[pl.ds(row * ROW_LEN + c * 16, 16)]
                    oa[row, pl.ds(c * 16, 16)] = plsc.load_gather(
                        table_v, [iv + bA]
                    )
                    ob[row, pl.ds(c * 16, 16)] = plsc.load_gather(
                        table_v, [iv + bB]
                    )

            pltpu.sync_copy(oa, out_hbm.at[hp, pl.ds(rg * 8, 8), :])
            pltpu.sync_copy(ob, out_hbm.at[hp + 8, pl.ds(rg * 8, 8), :])

        return carry

    lax.fori_loop(0, UNITS_PER_W, unit_body, 0, unroll=False)


def _slice_body(in_ref, out_ref):
    out_ref[...] = in_ref[:, :SEQ, :SEQ]


_slice_tc = pl.pallas_call(
    _slice_body,
    out_shape=jax.ShapeDtypeStruct((HEADS, SEQ, SEQ), jnp.float32),
    grid=(HEADS,),
    in_specs=[
        pl.BlockSpec((1, ROWS_PAD, ROW_LEN), lambda h: (h, 0, 0)),
    ],
    out_specs=pl.BlockSpec((1, SEQ, SEQ), lambda h: (h, 0, 0)),
)


def kernel(relative_position_bias_table, relative_position_index):
    flat_t = relative_position_bias_table.T.reshape(-1)
    idx_p = (
        jnp.zeros((ROWS_PAD, ROW_LEN), jnp.int32)
        .at[:SEQ, :SEQ]
        .set(relative_position_index)
        .reshape(-1)
    )
    padded = _rpb_sc(flat_t, idx_p)
    return _slice_tc(padded)


# speedup vs baseline: 21.2370x; 1.2313x over previous
import functools

import jax
import jax.numpy as jnp
from jax import lax
from jax.experimental import pallas as pl
from jax.experimental.pallas import tpu as pltpu
from jax.experimental.pallas import tpu_sc as plsc

HEADS = 16
SEQ = 1025
NREL = 3972
ROWS_PAD = 1032
ROW_LEN = 1040
RQ = ROWS_PAD // 4
UNITS = 2 * RQ
NW = 32
UNITS_PER_W = -(-UNITS // NW)
GROUPS = ROW_LEN // 16


@functools.partial(
    pl.kernel,
    out_type=jax.ShapeDtypeStruct((ROWS_PAD, HEADS, ROW_LEN), jnp.float32),
    mesh=plsc.VectorSubcoreMesh(core_axis_name="c", subcore_axis_name="s"),
    compiler_params=pltpu.CompilerParams(needs_layout_passes=False),
    scratch_types=[
        pltpu.VMEM((8 * NREL,), jnp.float32),
        pltpu.VMEM((4 * ROW_LEN,), jnp.int32),
        pltpu.VMEM((4, 8, ROW_LEN), jnp.float32),
    ],
)
def _rpb_sc(table_hbm, idx_hbm, out_hbm, table_v, idx_v, ob):
    cid = lax.axis_index("c")
    sid = lax.axis_index("s")
    wid = sid * 2 + cid

    hg = wid % 2
    pltpu.sync_copy(table_hbm.at[pl.ds(hg * (8 * NREL), 8 * NREL)], table_v)

    def unit_body(t, carry):
        u = wid + NW * t

        @pl.when(u < UNITS)
        def _():
            rq = u // 2
            pltpu.sync_copy(
                idx_hbm.at[pl.ds(rq * (4 * ROW_LEN), 4 * ROW_LEN)], idx_v
            )
            for row in range(4):

                @plsc.parallel_loop(0, GROUPS, unroll=4)
                def _g(c):
                    iv = idx_v[pl.ds(row * ROW_LEN + c * 16, 16)]
                    for h in range(8):
                        ob[row, h, pl.ds(c * 16, 16)] = plsc.load_gather(
                            table_v, [iv + h * NREL]
                        )

            pltpu.sync_copy(
                ob,
                out_hbm.at[
                    pl.ds(rq * 4, 4), pl.ds(pl.multiple_of(hg * 8, 8), 8), :
                ],
            )

        return carry

    lax.fori_loop(0, UNITS_PER_W, unit_body, 0, unroll=False)


def _slice_body(in_ref, out_ref):
    out_ref[...] = in_ref[:, :, :SEQ]


_slice_tc = pl.pallas_call(
    _slice_body,
    out_shape=jax.ShapeDtypeStruct((SEQ, HEADS, SEQ), jnp.float32),
    grid=(RQ // 2,),
    in_specs=[
        pl.BlockSpec((8, HEADS, ROW_LEN), lambda r: (r, 0, 0)),
    ],
    out_specs=pl.BlockSpec((8, HEADS, SEQ), lambda r: (r, 0, 0)),
)


def kernel(relative_position_bias_table, relative_position_index):
    flat_t = relative_position_bias_table.T.reshape(-1)
    idx_p = (
        jnp.zeros((ROWS_PAD, ROW_LEN), jnp.int32)
        .at[:SEQ, :SEQ]
        .set(relative_position_index)
        .reshape(-1)
    )
    padded = _rpb_sc(flat_t, idx_p)
    sliced = _slice_tc(padded)
    return jnp.transpose(sliced, (1, 0, 2))


# speedup vs baseline: 42.9547x; 2.0226x over previous
import functools

import jax
import jax.numpy as jnp
from jax import lax
from jax.experimental import pallas as pl
from jax.experimental.pallas import tpu as pltpu
from jax.experimental.pallas import tpu_sc as plsc

HEADS = 16
SEQ = 1025
NREL = 3972
ROW_LEN = 1040
RQ = 257
UNITS = 2 * RQ
NW = 32
UNITS_PER_W = -(-UNITS // NW)
GROUPS = ROW_LEN // 16


@functools.partial(
    pl.kernel,
    out_type=jax.ShapeDtypeStruct((SEQ, HEADS, SEQ), jnp.float32),
    mesh=plsc.VectorSubcoreMesh(core_axis_name="c", subcore_axis_name="s"),
    compiler_params=pltpu.CompilerParams(needs_layout_passes=False),
    scratch_types=[
        pltpu.VMEM((8 * NREL,), jnp.float32),
        pltpu.VMEM((4 * ROW_LEN,), jnp.int32),
        pltpu.VMEM((4, 8, SEQ), jnp.float32),
    ],
)
def _rpb_sc(table_hbm, idx_hbm, out_hbm, table_v, idx_v, ob):
    cid = lax.axis_index("c")
    sid = lax.axis_index("s")
    wid = sid * 2 + cid

    hg = wid % 2
    h0 = pl.multiple_of(hg * 8, 8)
    pltpu.sync_copy(table_hbm.at[pl.ds(hg * (8 * NREL), 8 * NREL)], table_v)

    def compute_rows(rq, nrows):
        pltpu.sync_copy(
            idx_hbm.at[pl.ds(rq * (4 * ROW_LEN), nrows * ROW_LEN)],
            idx_v.at[pl.ds(0, nrows * ROW_LEN)],
        )
        lane = lax.iota(jnp.int32, 16)
        last_col = jnp.full((16,), SEQ - 1, jnp.int32)
        last_mask = lane < 1
        for row in range(nrows):

            @plsc.parallel_loop(0, GROUPS - 1, unroll=4)
            def _g(c):
                iv = idx_v[pl.ds(row * ROW_LEN + c * 16, 16)]
                for h in range(8):
                    ob[row, h, pl.ds(c * 16, 16)] = plsc.load_gather(
                        table_v, [iv + h * NREL]
                    )

            ivl = idx_v[pl.ds(row * ROW_LEN + (SEQ - 1), 16)]
            row_idx = jnp.full((16,), row, jnp.int32)
            for h in range(8):
                vals = plsc.load_gather(table_v, [ivl + h * NREL])
                plsc.store_scatter(
                    ob,
                    [row_idx, jnp.full((16,), h, jnp.int32), last_col],
                    vals,
                    mask=last_mask,
                )

        pltpu.sync_copy(
            ob.at[pl.ds(0, nrows), :, :],
            out_hbm.at[pl.ds(rq * 4, nrows), pl.ds(h0, 8), :],
        )

    def unit_body(t, carry):
        u = wid + NW * t

        @pl.when(u < UNITS)
        def _():
            rq = u // 2

            @pl.when(rq < RQ - 1)
            def _():
                compute_rows(rq, 4)

            @pl.when(rq == RQ - 1)
            def _():
                compute_rows(rq, 1)

        return carry

    lax.fori_loop(0, UNITS_PER_W, unit_body, 0, unroll=False)


def kernel(relative_position_bias_table, relative_position_index):
    flat_t = relative_position_bias_table.T.reshape(-1)
    idx_p = (
        jnp.zeros((1028, ROW_LEN), jnp.int32)
        .at[:SEQ, :SEQ]
        .set(relative_position_index)
        .reshape(-1)
    )
    out = _rpb_sc(flat_t, idx_p)
    return jnp.transpose(out, (1, 0, 2))


# speedup vs baseline: 65.5147x; 1.5252x over previous
import functools

import jax
import jax.numpy as jnp
from jax import lax
from jax.experimental import pallas as pl
from jax.experimental.pallas import tpu as pltpu
from jax.experimental.pallas import tpu_sc as plsc

HEADS = 16
SEQ = 1025
NREL = 3972
ROW_LEN = 1040
NW = 32
UPW = 16
GROUPS = ROW_LEN // 16
IDX_ROWS = 1028


@functools.partial(
    pl.kernel,
    out_type=jax.ShapeDtypeStruct((SEQ, HEADS, SEQ), jnp.float32),
    mesh=plsc.VectorSubcoreMesh(core_axis_name="c", subcore_axis_name="s"),
    compiler_params=pltpu.CompilerParams(needs_layout_passes=False),
    scratch_types=[
        pltpu.VMEM((8 * NREL,), jnp.float32),
        pltpu.VMEM((4 * ROW_LEN,), jnp.int32),
        pltpu.VMEM((4 * ROW_LEN,), jnp.int32),
        pltpu.VMEM((4, 8, SEQ), jnp.float32),
        pltpu.VMEM((4, 8, SEQ), jnp.float32),
        pltpu.SemaphoreType.DMA,
        pltpu.SemaphoreType.DMA,
        pltpu.SemaphoreType.DMA,
        pltpu.SemaphoreType.DMA,
    ],
)
def _rpb_sc(
    table_hbm,
    idx_hbm,
    out_hbm,
    table_v,
    idx0,
    idx1,
    ob0,
    ob1,
    sem_i0,
    sem_i1,
    sem_o0,
    sem_o1,
):
    cid = lax.axis_index("c")
    sid = lax.axis_index("s")
    wid = sid * 2 + cid

    hg = wid % 2
    h0 = pl.multiple_of(hg * 8, 8)
    rq0 = wid // 2
    pltpu.sync_copy(table_hbm.at[pl.ds(hg * (8 * NREL), 8 * NREL)], table_v)

    lane = lax.iota(jnp.int32, 16)
    last_col = jnp.full((16,), SEQ - 1, jnp.int32)
    last_mask = lane < 1

    def idx_src(t):
        rq = rq0 + 16 * t
        return idx_hbm.at[pl.ds(rq * (4 * ROW_LEN), 4 * ROW_LEN)]

    def out_dst(t, nrows=4):
        rq = rq0 + 16 * t
        return out_hbm.at[pl.ds(rq * 4, nrows), pl.ds(h0, 8), :]

    def compute(idx_v, ob, nrows):
        for row in range(nrows):

            @plsc.parallel_loop(0, GROUPS - 1, unroll=8)
            def _g(c):
                iv = idx_v[pl.ds(row * ROW_LEN + c * 16, 16)]
                for h in range(8):
                    ob[row, h, pl.ds(c * 16, 16)] = plsc.load_gather(
                        table_v, [iv + h * NREL]
                    )

            ivl = idx_v[pl.ds(row * ROW_LEN + (SEQ - 1), 16)]
            row_idx = jnp.full((16,), row, jnp.int32)
            for h in range(8):
                vals = plsc.load_gather(table_v, [ivl + h * NREL])
                plsc.store_scatter(
                    ob,
                    [row_idx, jnp.full((16,), h, jnp.int32), last_col],
                    vals,
                    mask=last_mask,
                )

    def body(t, b, idx_b, idx_n, ob_b, sem_i_b, sem_i_n, sem_o_b):
        @pl.when(t + 1 < UPW)
        def _():
            pltpu.async_copy(idx_src(t + 1), idx_n, sem_i_n)

        pltpu.make_async_copy(idx_src(t), idx_b, sem_i_b).wait()

        @pl.when(t >= 2)
        def _():
            pltpu.make_async_copy(ob_b, out_dst(t - 2), sem_o_b).wait()

        compute(idx_b, ob_b, 4)
        pltpu.async_copy(ob_b, out_dst(t), sem_o_b)

    pltpu.async_copy(idx_src(0), idx0, sem_i0)

    def pair(k, carry):
        body(2 * k, 0, idx0, idx1, ob0, sem_i0, sem_i1, sem_o0)
        body(2 * k + 1, 1, idx1, idx0, ob1, sem_i1, sem_i0, sem_o1)
        return carry

    lax.fori_loop(0, UPW // 2, pair, 0, unroll=False)

    pltpu.make_async_copy(ob0, out_dst(UPW - 2), sem_o0).wait()
    pltpu.make_async_copy(ob1, out_dst(UPW - 1), sem_o1).wait()

    @pl.when(wid < 2)
    def _():
        pltpu.sync_copy(
            idx_hbm.at[pl.ds(1024 * ROW_LEN, ROW_LEN)],
            idx0.at[pl.ds(0, ROW_LEN)],
        )
        compute(idx0, ob0, 1)
        pltpu.sync_copy(
            ob0.at[pl.ds(0, 1), :, :],
            out_hbm.at[pl.ds(1024, 1), pl.ds(h0, 8), :],
        )


def kernel(relative_position_bias_table, relative_position_index):
    flat_t = relative_position_bias_table.T.reshape(-1)
    idx_p = (
        jnp.zeros((IDX_ROWS, ROW_LEN), jnp.int32)
        .at[:SEQ, :SEQ]
        .set(relative_position_index)
        .reshape(-1)
    )
    out = _rpb_sc(flat_t, idx_p)
    return jnp.transpose(out, (1, 0, 2))
